# baseline (speedup 1.0000x reference)
import functools
import math

import jax
import jax.numpy as jnp
from jax import lax
from jax.experimental import pallas as pl
from jax.experimental.pallas import tpu as pltpu
from jax.experimental.pallas import tpu_sc as plsc

D_MODEL = 64
SCALE = math.sqrt(D_MODEL)

_INFO = plsc.get_sparse_core_info()
_NC, _NS, _L = _INFO.num_cores, _INFO.num_subcores, _INFO.num_lanes
_NW = _NC * _NS

ROWS_PER_CHUNK = 2


def _make_emb(S: int, T: int):
    s_per_w = S // _NW
    nchunks = s_per_w // ROWS_PER_CHUNK
    chunk = ROWS_PER_CHUNK * T
    splits = []
    off = 0
    while off < T:
        n = min(128, T - off)
        splits.append((off, n))
        off += n
    assert nchunks % 2 == 0 and nchunks >= 2

    mesh = plsc.VectorSubcoreMesh(core_axis_name="c", subcore_axis_name="s")

    @functools.partial(
        pl.kernel,
        mesh=mesh,
        out_type=jax.ShapeDtypeStruct((S, T, D_MODEL), jnp.float32),
        scratch_types=[
            pltpu.VMEM((s_per_w, T), jnp.int32),
            pltpu.VMEM((2, ROWS_PER_CHUNK * T, D_MODEL), jnp.float32),
            pltpu.SemaphoreType.DMA,
            pltpu.SemaphoreType.DMA,
        ],
        compiler_params=pltpu.CompilerParams(use_tc_tiling_on_sc=False),
    )
    def emb(x_hbm, lut_hbm, out_hbm, idx_v, rows_v, sem_g, sem_s):
        wid = lax.axis_index("s") * _NC + lax.axis_index("c")
        s_base = pl.multiple_of(wid * s_per_w, s_per_w)
        pltpu.sync_copy(x_hbm.at[pl.ds(s_base, s_per_w)], idx_v)

        def fire_gather(ci, buf):
            dst = rows_v.at[buf]
            for r in range(ROWS_PER_CHUNK):
                for off, n in splits:
                    pltpu.async_copy(
                        lut_hbm.at[idx_v.at[ci * ROWS_PER_CHUNK + r]
                                   .at[pl.ds(off, n)]],
                        dst.at[pl.ds(r * T + off, n)],
                        sem_g,
                    )

        def wait_gather(ci, buf):
            dst = rows_v.at[buf]
            for r in range(ROWS_PER_CHUNK):
                for off, n in splits:
                    pltpu.make_async_copy(
                        lut_hbm.at[idx_v.at[ci * ROWS_PER_CHUNK + r]
                                   .at[pl.ds(off, n)]],
                        dst.at[pl.ds(r * T + off, n)],
                        sem_g,
                    ).wait()

        def store_copies(ci, buf):
            s0 = pl.multiple_of(s_base + ci * ROWS_PER_CHUNK, ROWS_PER_CHUNK)
            return [
                pltpu.make_async_copy(
                    rows_v.at[buf].at[pl.ds(r * T, T)],
                    out_hbm.at[s0 + r],
                    sem_s,
                )
                for r in range(ROWS_PER_CHUNK)
            ]

        fire_gather(0, 0)

        @pl.loop(0, nchunks, step=2)
        def _outer(ci0):
            for b in range(2):
                ci = ci0 + b
                nb = 1 - b

                @pl.when(ci >= 1)
                def _wait_prev_store():
                    for c in store_copies(ci - 1, nb):
                        c.wait()

                @pl.when(ci + 1 < nchunks)
                def _fire_next_gather():
                    fire_gather(ci + 1, nb)

                wait_gather(ci, b)

                flat = rows_v.at[b]

                @pl.loop(0, chunk, unroll=8)
                def _scale(r):
                    for j in range(D_MODEL // _L):
                        sl = pl.ds(j * _L, _L)
                        flat[r, sl] = flat[r, sl] * jnp.float32(SCALE)

                for c in store_copies(ci, b):
                    c.start()

        for c in store_copies(nchunks - 1, 1):
            c.wait()

    return emb


def kernel(x, lut):
    return _make_emb(x.shape[0], x.shape[1])(x, lut)
